# baseline (speedup 1.0000x reference)
import functools

import jax
import jax.numpy as jnp
from jax import lax
from jax.experimental import pallas as pl
from jax.experimental.pallas import tpu as pltpu
from jax.experimental.pallas import tpu_sc as plsc

D = 128
CH = 128
NC = 2
NS = 16
NW = NC * NS
L = 16


def _sc_dot_gather(n_edges):
    assert n_edges % (8 * CH * NW) == 0
    n_chunks = n_edges // (CH * NW)
    n_pairs = n_chunks // 2
    mesh = plsc.VectorSubcoreMesh(core_axis_name="c", subcore_axis_name="s")

    @functools.partial(
        pl.kernel,
        mesh=mesh,
        compiler_params=pltpu.CompilerParams(needs_layout_passes=False),
        out_type=jax.ShapeDtypeStruct((n_edges,), jnp.float32),
        scratch_types=[
            pltpu.VMEM((n_chunks, CH), jnp.int32),
            pltpu.VMEM((n_chunks, CH), jnp.int32),
            pltpu.VMEM((CH, D), jnp.float32),
            pltpu.VMEM((CH, D), jnp.float32),
            pltpu.VMEM((CH, D), jnp.float32),
            pltpu.VMEM((CH, D), jnp.float32),
            pltpu.VMEM((CH,), jnp.float32),
            pltpu.SemaphoreType.DMA,
            pltpu.SemaphoreType.DMA,
            pltpu.SemaphoreType.DMA,
            pltpu.SemaphoreType.DMA,
        ],
    )
    def k(xu, xb, iu, ib, out, idxu, idxb, ru0, ru1, rb0, rb1, ov,
          su0, su1, sb0, sb1):
        ru = [ru0, ru1]
        rb = [rb0, rb1]
        su = [su0, su1]
        sb = [sb0, sb1]
        wid = lax.axis_index("s") * NC + lax.axis_index("c")
        wbase = wid * n_chunks
        iota = lax.iota(jnp.int32, L)
        rids = [g * L + iota for g in range(CH // L)]

        pltpu.sync_copy(iu.at[pl.ds(wbase, n_chunks)], idxu)
        pltpu.sync_copy(ib.at[pl.ds(wbase, n_chunks)], idxb)

        def gathers(c, b):
            cu = pltpu.make_async_copy(xu.at[idxu.at[c]], ru[b], su[b])
            cb = pltpu.make_async_copy(xb.at[idxb.at[c]], rb[b], sb[b])
            cu.start()
            cb.start()

        def wait_gathers(c, b):
            pltpu.make_async_copy(xu.at[idxu.at[c]], ru[b], su[b]).wait()
            pltpu.make_async_copy(xb.at[idxb.at[c]], rb[b], sb[b]).wait()

        def do_chunk(c, b):
            @pl.when(c + 1 < n_chunks)
            def _():
                gathers(c + 1, 1 - b)

            wait_gathers(c, b)

            def dbody(dd, accs):
                col = (iota + dd) & (D - 1)
                return tuple(
                    accs[g]
                    + plsc.load_gather(ru[b], [rids[g], col])
                    * plsc.load_gather(rb[b], [rids[g], col])
                    for g in range(CH // L)
                )

            zero = jnp.zeros((L,), jnp.float32)
            accs = lax.fori_loop(0, D, dbody, tuple(zero for _ in range(CH // L)))
            for g in range(CH // L):
                ov[pl.ds(g * L, L)] = accs[g]
            pltpu.sync_copy(ov, out.at[pl.ds((wbase + c) * CH, CH)])

        gathers(0, 0)

        def pair_body(i, carry):
            for b in range(2):
                do_chunk(i * 2 + b, b)
            return carry

        lax.fori_loop(0, n_pairs, pair_body, 0)

    return k


def kernel(x_user, x_book, edge_label_index):
    eli = edge_label_index.astype(jnp.int32)
    n = eli.shape[1]
    step = 8 * CH * NW
    n_pad = ((n + step - 1) // step) * step
    iu = jnp.pad(eli[0], (0, n_pad - n)).reshape(n_pad // CH, CH)
    ib = jnp.pad(eli[1], (0, n_pad - n)).reshape(n_pad // CH, CH)
    out = _sc_dot_gather(n_pad)(x_user, x_book, iu, ib)
    return out[:n]


# speedup vs baseline: 2.9770x; 2.9770x over previous
import functools

import jax
import jax.numpy as jnp
from jax import lax
from jax.experimental import pallas as pl
from jax.experimental.pallas import tpu as pltpu
from jax.experimental.pallas import tpu_sc as plsc

D = 128
CH = 64
DEPTH = 6
NC = 2
NS = 16
NW = NC * NS
L = 16
G = CH // L


def _sc_dot_gather(n_edges):
    assert n_edges % (DEPTH * CH * NW) == 0
    n_chunks = n_edges // (CH * NW)
    kah = DEPTH - 2

    mesh = plsc.VectorSubcoreMesh(core_axis_name="c", subcore_axis_name="s")

    @functools.partial(
        pl.kernel,
        mesh=mesh,
        compiler_params=pltpu.CompilerParams(needs_layout_passes=False),
        out_type=jax.ShapeDtypeStruct((n_edges,), jnp.float32),
        scratch_types=(
            [pltpu.VMEM((CH, D), jnp.float32) for _ in range(2 * DEPTH)]
            + [pltpu.VMEM((CH,), jnp.int32) for _ in range(2 * DEPTH)]
            + [pltpu.VMEM((CH,), jnp.float32)]
            + [pltpu.SemaphoreType.DMA for _ in range(4 * DEPTH)]
        ),
    )
    def k(xu, xb, iu, ib, out, *bufs):
        ru = list(bufs[0:DEPTH])
        rb = list(bufs[DEPTH:2 * DEPTH])
        ivu = list(bufs[2 * DEPTH:3 * DEPTH])
        ivb = list(bufs[3 * DEPTH:4 * DEPTH])
        ov = bufs[4 * DEPTH]
        sems = bufs[4 * DEPTH + 1:]
        su = list(sems[0:DEPTH])
        sb = list(sems[DEPTH:2 * DEPTH])
        qu = list(sems[2 * DEPTH:3 * DEPTH])
        qb = list(sems[3 * DEPTH:4 * DEPTH])

        wid = lax.axis_index("s") * NC + lax.axis_index("c")
        wbase = wid * n_chunks * CH
        iota = lax.iota(jnp.int32, L)
        rids = [g * L + iota for g in range(G)]

        def idx_copies(c, r):
            pltpu.make_async_copy(iu.at[pl.ds(wbase + c * CH, CH)], ivu[r], qu[r]).start()
            pltpu.make_async_copy(ib.at[pl.ds(wbase + c * CH, CH)], ivb[r], qb[r]).start()

        def wait_idx(c, r):
            pltpu.make_async_copy(iu.at[pl.ds(wbase + c * CH, CH)], ivu[r], qu[r]).wait()
            pltpu.make_async_copy(ib.at[pl.ds(wbase + c * CH, CH)], ivb[r], qb[r]).wait()

        def gathers(r):
            pltpu.make_async_copy(xu.at[ivu[r]], ru[r], su[r]).start()
            pltpu.make_async_copy(xb.at[ivb[r]], rb[r], sb[r]).start()

        def wait_gathers(r):
            pltpu.make_async_copy(xu.at[ivu[r]], ru[r], su[r]).wait()
            pltpu.make_async_copy(xb.at[ivb[r]], rb[r], sb[r]).wait()

        for c in range(kah + 1):
            idx_copies(c, c % DEPTH)
        for c in range(kah):
            wait_idx(c, c % DEPTH)
            gathers(c % DEPTH)

        def do_chunk(c, r):
            @pl.when(c + kah < n_chunks)
            def _():
                wait_idx(c + kah, (r + kah) % DEPTH)
                gathers((r + kah) % DEPTH)

            @pl.when(c + kah + 1 < n_chunks)
            def _():
                idx_copies(c + kah + 1, (r + kah + 1) % DEPTH)

            wait_gathers(r)

            def dbody(dd, accs):
                col = (iota + dd) & (D - 1)
                return tuple(
                    accs[g]
                    + plsc.load_gather(ru[r], [rids[g], col])
                    * plsc.load_gather(rb[r], [rids[g], col])
                    for g in range(G)
                )

            zero = jnp.zeros((L,), jnp.float32)
            accs = lax.fori_loop(0, D, dbody, tuple(zero for _ in range(G)))
            for g in range(G):
                ov[pl.ds(g * L, L)] = accs[g]
            pltpu.sync_copy(ov, out.at[pl.ds(wbase + c * CH, CH)])

        def ring_body(i, carry):
            for b in range(DEPTH):
                do_chunk(i * DEPTH + b, b)
            return carry

        lax.fori_loop(0, n_chunks // DEPTH, ring_body, 0)

    return k


def kernel(x_user, x_book, edge_label_index):
    eli = edge_label_index.astype(jnp.int32)
    n = eli.shape[1]
    step = DEPTH * CH * NW
    n_pad = ((n + step - 1) // step) * step
    iu = jnp.pad(eli[0], (0, n_pad - n))
    ib = jnp.pad(eli[1], (0, n_pad - n))
    out = _sc_dot_gather(n_pad)(x_user, x_book, iu, ib)
    return out[:n]
